# Initial kernel scaffold; baseline (speedup 1.0000x reference)
import jax
import jax.numpy as jnp
from jax.experimental import pallas as pl


def kernel(embeddings, labels, batch_size):
    raise NotImplementedError("write your pallas kernel here")


# speedup vs baseline: 91.2510x; 91.2510x over previous
import jax
import jax.numpy as jnp
from jax.experimental import pallas as pl
from jax.experimental.pallas import tpu as pltpu

_M = 0.4
_GAMMA = 80.0
_NEG_BIG = -1e30


def _loss_body(e_ref, lab_row_ref, lab_col_ref, filt_col_ref, out_ref):
    E = e_ref[...]
    lab_row = lab_row_ref[...]
    lab_col = lab_col_ref[...]
    filt_col = filt_col_ref[...]
    n = E.shape[0]

    S = jax.lax.dot_general(
        E, E, (((1,), (1,)), ((), ())), preferred_element_type=jnp.float32
    )

    same = lab_col == lab_row
    row_i = jax.lax.broadcasted_iota(jnp.int32, (n, n), 0)
    col_j = jax.lax.broadcasted_iota(jnp.int32, (n, n), 1)
    pos = same & (row_i != col_j) & (filt_col > 0)
    neg = jnp.logical_not(same)

    alpha_p = jnp.maximum((1.0 + _M) - S, 0.0)
    alpha_n = jnp.maximum(S + _M, 0.0)
    logit_p = -_GAMMA * alpha_p * (S - (1.0 - _M))
    logit_n = _GAMMA * alpha_n * (S - _M)

    lp = jnp.where(pos, logit_p, _NEG_BIG)
    ln = jnp.where(neg, logit_n, _NEG_BIG)
    m_p = jnp.max(lp, axis=1, keepdims=True)
    m_n = jnp.max(ln, axis=1, keepdims=True)
    sum_p = jnp.sum(jnp.where(pos, jnp.exp(logit_p - m_p), 0.0), axis=1,
                    keepdims=True)
    sum_n = jnp.sum(jnp.where(neg, jnp.exp(logit_n - m_n), 0.0), axis=1,
                    keepdims=True)
    cnt_p = jnp.sum(pos.astype(jnp.float32), axis=1, keepdims=True)
    cnt_n = jnp.sum(neg.astype(jnp.float32), axis=1, keepdims=True)

    valid = (filt_col > 0) & (cnt_p > 0) & (cnt_n > 0)
    lse = (m_p + jnp.log(sum_p) + jnp.log(cnt_n)
           + m_n + jnp.log(sum_n) + jnp.log(cnt_p))
    term = jnp.where(
        valid,
        jnp.maximum(lse, 0.0) + jnp.log1p(jnp.exp(-jnp.abs(lse))),
        0.0,
    )
    total = jnp.sum(term)
    cnt = jnp.sum(valid.astype(jnp.float32))
    out_ref[...] = jnp.where(cnt > 0, total / cnt, 0.0).reshape(1, 1)


def kernel(embeddings, labels, batch_size):
    n = embeddings.shape[0]
    ar = jnp.arange(n, dtype=jnp.int32)
    bs = jnp.asarray(batch_size, jnp.int32)
    filt = (((ar % 4 == 0) & (ar < bs)) | (ar > bs)).astype(jnp.int32)
    lab = labels.astype(jnp.int32)
    out = pl.pallas_call(
        _loss_body,
        out_shape=jax.ShapeDtypeStruct((1, 1), jnp.float32),
    )(embeddings.astype(jnp.float32), lab.reshape(1, n), lab.reshape(n, 1),
      filt.reshape(n, 1))
    return out[0, 0]
